# Initial kernel scaffold; baseline (speedup 1.0000x reference)
import jax
import jax.numpy as jnp
from jax.experimental import pallas as pl


def kernel(output, mask, ind, target):
    raise NotImplementedError("write your pallas kernel here")


# speedup vs baseline: 1.4672x; 1.4672x over previous
import functools

import jax
import jax.numpy as jnp
from jax import lax
from jax.experimental import pallas as pl
from jax.experimental.pallas import tpu as pltpu
from jax.experimental.pallas import tpu_sc as plsc

B, D, H, W = 64, 4, 256, 256
K = 128
HW = H * W
NC, NS, L = 2, 16, 16
NW = NC * NS
BPW = B // NW
KCH = K // L


@functools.partial(
    pl.kernel,
    mesh=plsc.VectorSubcoreMesh(core_axis_name="c", subcore_axis_name="s"),
    out_type=jax.ShapeDtypeStruct((NW, 2 * L), jnp.float32),
    scratch_types=[
        pltpu.VMEM((BPW * K,), jnp.int32),
        pltpu.VMEM((BPW * K,), jnp.int32),
        pltpu.VMEM((BPW * D, K), jnp.int32),
        pltpu.VMEM((BPW * D, K), jnp.int32),
        pltpu.VMEM((BPW * D, K), jnp.float32),
        pltpu.VMEM((BPW * D, K), jnp.float32),
        pltpu.VMEM((2 * L,), jnp.float32),
        pltpu.SemaphoreType.DMA,
    ],
)
def _reg_loss_sc(feat, ind, mask, target, out, ind_v, mask_v, idx_v, tix_v,
                 pred_v, tgt_v, obuf, sem):
    cid = lax.axis_index("c")
    sid = lax.axis_index("s")
    wid = sid * NC + cid
    b0 = wid * BPW

    pltpu.sync_copy(ind.at[pl.ds(b0 * K, BPW * K)], ind_v)
    pltpu.sync_copy(mask.at[pl.ds(b0 * K, BPW * K)], mask_v)

    ki = lax.iota(jnp.int32, L)
    for bi in range(BPW):
        for d in range(D):
            base = ((b0 + bi) * D + d) * HW
            tbase = (b0 + bi) * K * D + d
            for c in range(KCH):
                chunk = ind_v[pl.ds(bi * K + c * L, L)]
                idx_v[bi * D + d, pl.ds(c * L, L)] = chunk + base
                tix_v[bi * D + d, pl.ds(c * L, L)] = (ki + c * L) * D + tbase

    copies = [
        pltpu.async_copy(feat.at[idx_v.at[r]], pred_v.at[r], sem)
        for r in range(BPW * D)
    ] + [
        pltpu.async_copy(target.at[tix_v.at[r]], tgt_v.at[r], sem)
        for r in range(BPW * D)
    ]
    for cp in copies:
        cp.wait()

    acc = jnp.zeros((L,), jnp.float32)
    msum = jnp.zeros((L,), jnp.float32)
    for bi in range(BPW):
        for c in range(KCH):
            m = mask_v[pl.ds(bi * K + c * L, L)].astype(jnp.float32)
            msum = msum + m
            for d in range(D):
                p = pred_v[bi * D + d, pl.ds(c * L, L)]
                t = tgt_v[bi * D + d, pl.ds(c * L, L)]
                df = (p - t) * m
                ad = jnp.abs(df)
                acc = acc + jnp.where(ad < 1.0, 0.5 * df * df, ad - 0.5)

    obuf[pl.ds(0, L)] = acc
    obuf[pl.ds(L, L)] = msum
    pltpu.sync_copy(obuf, out.at[wid])


def kernel(output, mask, ind, target):
    feat = output.astype(jnp.float32).reshape(-1)
    indf = ind.astype(jnp.int32).reshape(-1)
    maskf = mask.astype(jnp.int32).reshape(-1)
    tgtf = target.astype(jnp.float32).reshape(-1)
    parts = _reg_loss_sc(feat, indf, maskf, tgtf)
    loss = jnp.sum(parts[:, :L])
    num = jnp.sum(parts[:, L:])
    return loss / (num + 0.0001)


# speedup vs baseline: 2.5948x; 1.7685x over previous
import functools

import jax
import jax.numpy as jnp
from jax import lax
from jax.experimental import pallas as pl
from jax.experimental.pallas import tpu as pltpu
from jax.experimental.pallas import tpu_sc as plsc

B, D, H, W = 64, 4, 256, 256
K = 128
HW = H * W
NC, NS, L = 2, 16, 16
NW = NC * NS
BPW = B // NW
KCH = K // L
NR = BPW * D


@functools.partial(
    pl.kernel,
    mesh=plsc.VectorSubcoreMesh(core_axis_name="c", subcore_axis_name="s"),
    out_type=jax.ShapeDtypeStruct((NW, 2 * L), jnp.float32),
    compiler_params=pltpu.CompilerParams(needs_layout_passes=False),
    scratch_types=[
        pltpu.VMEM((BPW * K,), jnp.int32),
        pltpu.VMEM((BPW * K,), jnp.int32),
        pltpu.VMEM((NR, K), jnp.int32),
        pltpu.VMEM((K, W), jnp.float32),
        pltpu.VMEM((K, W), jnp.float32),
        pltpu.VMEM((NR, K), jnp.float32),
        pltpu.VMEM((BPW * K * D,), jnp.float32),
        pltpu.VMEM((2 * L,), jnp.float32),
        pltpu.SemaphoreType.DMA,
        pltpu.SemaphoreType.DMA,
    ],
)
def _reg_loss_sc(feat2, ind, mask, target, out, ind_v, mask_v, ridx_v,
                 rb0, rb1, pred_v, tgt_v, obuf, sem0, sem1):
    cid = lax.axis_index("c")
    sid = lax.axis_index("s")
    wid = sid * NC + cid
    b0 = wid * BPW

    pltpu.sync_copy(ind.at[pl.ds(b0 * K, BPW * K)], ind_v)
    pltpu.sync_copy(mask.at[pl.ds(b0 * K, BPW * K)], mask_v)
    pltpu.sync_copy(target.at[pl.ds(b0 * K * D, BPW * K * D)], tgt_v)

    ki = lax.iota(jnp.int32, L)
    for bi in range(BPW):
        for c in range(KCH):
            chunk = ind_v[pl.ds(bi * K + c * L, L)]
            rin = chunk >> 8
            for d in range(D):
                rbase = ((b0 + bi) * D + d) * H
                ridx_v[bi * D + d, pl.ds(c * L, L)] = rin + rbase

    bufs = [rb0, rb1]
    sems = [sem0, sem1]
    pend = [
        pltpu.async_copy(feat2.at[ridx_v.at[r]], bufs[r], sems[r])
        for r in range(2)
    ]
    for r in range(NR):
        s = r % 2
        bi = r // D
        pend[s].wait()
        for c in range(KCH):
            chunk = ind_v[pl.ds(bi * K + c * L, L)]
            rows = ki + c * L
            cols = chunk & 255
            pred_v[r, pl.ds(c * L, L)] = plsc.load_gather(bufs[s], [rows, cols])
        if r + 2 < NR:
            pend[s] = pltpu.async_copy(feat2.at[ridx_v.at[r + 2]], bufs[s], sems[s])

    acc = jnp.zeros((L,), jnp.float32)
    msum = jnp.zeros((L,), jnp.float32)
    for bi in range(BPW):
        for c in range(KCH):
            m = mask_v[pl.ds(bi * K + c * L, L)].astype(jnp.float32)
            msum = msum + m
            for d in range(D):
                p = pred_v[bi * D + d, pl.ds(c * L, L)]
                t = plsc.load_gather(tgt_v, [(bi * K + c * L + ki) * D + d])
                df = (p - t) * m
                ad = jnp.abs(df)
                acc = acc + jnp.where(ad < 1.0, 0.5 * df * df, ad - 0.5)

    obuf[pl.ds(0, L)] = acc
    obuf[pl.ds(L, L)] = msum
    pltpu.sync_copy(obuf, out.at[wid])


def kernel(output, mask, ind, target):
    feat2 = output.astype(jnp.float32).reshape(B * D * H, W)
    indf = ind.astype(jnp.int32).reshape(-1)
    maskf = mask.astype(jnp.int32).reshape(-1)
    tgtf = target.astype(jnp.float32).reshape(-1)
    parts = _reg_loss_sc(feat2, indf, maskf, tgtf)
    loss = jnp.sum(parts[:, :L])
    num = jnp.sum(parts[:, L:])
    return loss / (num + 0.0001)


# speedup vs baseline: 2.6357x; 1.0158x over previous
import functools

import jax
import jax.numpy as jnp
from jax import lax
from jax.experimental import pallas as pl
from jax.experimental.pallas import tpu as pltpu
from jax.experimental.pallas import tpu_sc as plsc

B, D, H, W = 64, 4, 256, 256
K = 128
HW = H * W
NC, NS, L = 2, 16, 16
NW = NC * NS
BPW = B // NW
KCH = K // L
NR = BPW * D


@functools.partial(
    pl.kernel,
    mesh=plsc.VectorSubcoreMesh(core_axis_name="c", subcore_axis_name="s"),
    out_type=jax.ShapeDtypeStruct((NW, 2 * L), jnp.float32),
    compiler_params=pltpu.CompilerParams(needs_layout_passes=False),
    scratch_types=[
        pltpu.VMEM((BPW * K,), jnp.int32),
        pltpu.VMEM((BPW * K,), jnp.int32),
        pltpu.VMEM((NR, K), jnp.int32),
        pltpu.VMEM((K, W), jnp.float32),
        pltpu.VMEM((K, W), jnp.float32),
        pltpu.VMEM((K, W), jnp.float32),
        pltpu.VMEM((NR, K), jnp.float32),
        pltpu.VMEM((BPW * K * D,), jnp.float32),
        pltpu.VMEM((2 * L,), jnp.float32),
        pltpu.SemaphoreType.DMA,
        pltpu.SemaphoreType.DMA,
        pltpu.SemaphoreType.DMA,
    ],
)
def _reg_loss_sc(feat2, ind, mask, target, out, ind_v, mask_v, ridx_v,
                 rb0, rb1, rb2, pred_v, tgt_v, obuf, sem0, sem1, sem2):
    cid = lax.axis_index("c")
    sid = lax.axis_index("s")
    wid = sid * NC + cid
    b0 = wid * BPW

    pltpu.sync_copy(ind.at[pl.ds(b0 * K, BPW * K)], ind_v)
    pltpu.sync_copy(mask.at[pl.ds(b0 * K, BPW * K)], mask_v)
    pltpu.sync_copy(target.at[pl.ds(b0 * K * D, BPW * K * D)], tgt_v)

    ki = lax.iota(jnp.int32, L)
    for bi in range(BPW):
        for c in range(KCH):
            chunk = ind_v[pl.ds(bi * K + c * L, L)]
            rin = chunk >> 8
            for d in range(D):
                rbase = ((b0 + bi) * D + d) * H
                ridx_v[bi * D + d, pl.ds(c * L, L)] = rin + rbase

    bufs = [rb0, rb1, rb2]
    sems = [sem0, sem1, sem2]
    NB = 3
    pend = [
        pltpu.async_copy(feat2.at[ridx_v.at[r]], bufs[r], sems[r])
        for r in range(NB)
    ]
    for r in range(NR):
        s = r % NB
        bi = r // D
        pend[s].wait()
        for c in range(KCH):
            chunk = ind_v[pl.ds(bi * K + c * L, L)]
            rows = ki + c * L
            cols = chunk & 255
            pred_v[r, pl.ds(c * L, L)] = plsc.load_gather(bufs[s], [rows, cols])
        if r + NB < NR:
            pend[s] = pltpu.async_copy(feat2.at[ridx_v.at[r + NB]], bufs[s], sems[s])

    acc = jnp.zeros((L,), jnp.float32)
    msum = jnp.zeros((L,), jnp.float32)
    for bi in range(BPW):
        for c in range(KCH):
            m = mask_v[pl.ds(bi * K + c * L, L)].astype(jnp.float32)
            msum = msum + m
            for d in range(D):
                p = pred_v[bi * D + d, pl.ds(c * L, L)]
                t = plsc.load_gather(tgt_v, [(bi * K + c * L + ki) * D + d])
                df = (p - t) * m
                ad = jnp.abs(df)
                acc = acc + jnp.where(ad < 1.0, 0.5 * df * df, ad - 0.5)

    obuf[pl.ds(0, L)] = acc
    obuf[pl.ds(L, L)] = msum
    pltpu.sync_copy(obuf, out.at[wid])


def kernel(output, mask, ind, target):
    feat2 = output.astype(jnp.float32).reshape(B * D * H, W)
    indf = ind.astype(jnp.int32).reshape(-1)
    maskf = mask.astype(jnp.int32).reshape(-1)
    tgtf = target.astype(jnp.float32).reshape(-1)
    parts = _reg_loss_sc(feat2, indf, maskf, tgtf)
    loss = jnp.sum(parts[:, :L])
    num = jnp.sum(parts[:, L:])
    return loss / (num + 0.0001)
